# baseline (speedup 1.0000x reference)
import functools

import jax
import jax.numpy as jnp
from jax import lax
from jax.experimental import pallas as pl
from jax.experimental.pallas import tpu as pltpu
from jax.experimental.pallas import tpu_sc as plsc

N_NODES = 10000
N_EDGES = 320000
D_IN = 128

NC = 2
NS = 16
NW = NC * NS
CHUNK = 128
N_PAD = 10112
RPT = N_PAD // NS
EPT_CHUNKS = 84
E_PAD = NW * EPT_CHUNKS * CHUNK


def _sc_scatter(hw, zeros, src_r, dst_r, d, nbuf, lk):
    mesh = plsc.VectorSubcoreMesh(core_axis_name="c", subcore_axis_name="s")

    @functools.partial(
        pl.kernel,
        out_type=jax.ShapeDtypeStruct((NC, N_PAD, d), jnp.float32),
        mesh=mesh,
        scratch_types=[
            pltpu.VMEM((EPT_CHUNKS, CHUNK), jnp.int32),
            pltpu.VMEM((EPT_CHUNKS, CHUNK), jnp.int32),
            pltpu.VMEM((nbuf, CHUNK, d), jnp.float32),
            pltpu.VMEM_SHARED((N_PAD, d), jnp.float32),
            pltpu.SemaphoreType.DMA((nbuf,)),
            pltpu.SemaphoreType.DMA((nbuf,)),
        ],
        compiler_params=pltpu.CompilerParams(use_tc_tiling_on_sc=False),
    )
    def k(hw_hbm, z_hbm, src_hbm, dst_hbm, out_hbm,
          src_v, dst_v, rows_v, acc_sh, gsem, ssem):
        c = lax.axis_index("c")
        s = lax.axis_index("s")
        wid = s * NC + c
        pltpu.sync_copy(src_hbm.at[wid], src_v)
        pltpu.sync_copy(dst_hbm.at[wid], dst_v)
        pltpu.sync_copy(z_hbm.at[pl.ds(s * RPT, RPT)],
                        acc_sh.at[pl.ds(s * RPT, RPT)])
        plsc.subcore_barrier()

        def gissue(j, slot):
            pltpu.async_copy(hw_hbm.at[src_v.at[j]], rows_v.at[slot],
                             gsem.at[slot])

        def gwait(j, slot):
            pltpu.make_async_copy(hw_hbm.at[src_v.at[j]], rows_v.at[slot],
                                  gsem.at[slot]).wait()

        def sissue(j, slot):
            pltpu.async_copy(rows_v.at[slot], acc_sh.at[dst_v.at[j]],
                             ssem.at[slot], add=True)

        def swait(j, slot):
            pltpu.make_async_copy(rows_v.at[slot], acc_sh.at[dst_v.at[j]],
                                  ssem.at[slot]).wait()

        for jj in range(lk):
            gissue(jj, jj % nbuf)
        for jj in range(nbuf):
            if jj >= nbuf - lk:
                swait(jj + lk - nbuf, (jj + lk) % nbuf)
            gissue(jj + lk, (jj + lk) % nbuf)
            gwait(jj, jj % nbuf)
            sissue(jj, jj % nbuf)

        @pl.loop(nbuf, EPT_CHUNKS - nbuf, step=nbuf)
        def _(j0):
            for b in range(nbuf):
                j = j0 + b
                swait(j + lk - nbuf, (b + lk) % nbuf)
                gissue(j + lk, (b + lk) % nbuf)
                gwait(j, b)
                sissue(j, b)

        for jj in range(EPT_CHUNKS - nbuf, EPT_CHUNKS):
            swait(jj + lk - nbuf, (jj + lk) % nbuf)
            if jj + lk < EPT_CHUNKS:
                gissue(jj + lk, (jj + lk) % nbuf)
            gwait(jj, jj % nbuf)
            sissue(jj, jj % nbuf)
        for cc in range(EPT_CHUNKS - (nbuf - lk), EPT_CHUNKS):
            swait(cc, cc % nbuf)

        plsc.subcore_barrier()
        pltpu.sync_copy(acc_sh.at[pl.ds(s * RPT, RPT)],
                        out_hbm.at[c].at[pl.ds(s * RPT, RPT)])

    return k(hw, zeros, src_r, dst_r)


def _sc_scatter_fsplit(hw2, zeros, src_r, dst_r, nbuf, lk):
    dh = hw2.shape[2]
    n_chunks = src_r.shape[1]
    mesh = plsc.VectorSubcoreMesh(core_axis_name="c", subcore_axis_name="s")

    @functools.partial(
        pl.kernel,
        out_type=jax.ShapeDtypeStruct((NC, N_PAD, dh), jnp.float32),
        mesh=mesh,
        scratch_types=[
            pltpu.VMEM((n_chunks, CHUNK), jnp.int32),
            pltpu.VMEM((n_chunks, CHUNK), jnp.int32),
            pltpu.VMEM((nbuf, CHUNK, dh), jnp.float32),
            pltpu.VMEM_SHARED((N_PAD, dh), jnp.float32),
            pltpu.SemaphoreType.DMA((nbuf,)),
            pltpu.SemaphoreType.DMA((nbuf,)),
        ],
        compiler_params=pltpu.CompilerParams(use_tc_tiling_on_sc=False),
    )
    def k(hw_hbm, z_hbm, src_hbm, dst_hbm, out_hbm,
          src_v, dst_v, rows_v, acc_sh, gsem, ssem):
        c = lax.axis_index("c")
        s = lax.axis_index("s")
        table = hw_hbm.at[c]
        pltpu.sync_copy(src_hbm.at[s], src_v)
        pltpu.sync_copy(dst_hbm.at[s], dst_v)
        pltpu.sync_copy(z_hbm.at[pl.ds(s * RPT, RPT)],
                        acc_sh.at[pl.ds(s * RPT, RPT)])
        plsc.subcore_barrier()

        def gissue(j, slot):
            pltpu.async_copy(table.at[src_v.at[j]], rows_v.at[slot],
                             gsem.at[slot])

        def gwait(j, slot):
            pltpu.make_async_copy(table.at[src_v.at[j]], rows_v.at[slot],
                                  gsem.at[slot]).wait()

        def sissue(j, slot):
            pltpu.async_copy(rows_v.at[slot], acc_sh.at[dst_v.at[j]],
                             ssem.at[slot], add=True)

        def swait(j, slot):
            pltpu.make_async_copy(rows_v.at[slot], acc_sh.at[dst_v.at[j]],
                                  ssem.at[slot]).wait()

        for jj in range(lk):
            gissue(jj, jj % nbuf)
        for jj in range(nbuf):
            if jj >= nbuf - lk:
                swait(jj + lk - nbuf, (jj + lk) % nbuf)
            gissue(jj + lk, (jj + lk) % nbuf)
            gwait(jj, jj % nbuf)
            sissue(jj, jj % nbuf)

        @pl.loop(nbuf, n_chunks - nbuf, step=nbuf)
        def _(j0):
            for b in range(nbuf):
                j = j0 + b
                swait(j + lk - nbuf, (b + lk) % nbuf)
                gissue(j + lk, (b + lk) % nbuf)
                gwait(j, b)
                sissue(j, b)

        for jj in range(n_chunks - nbuf, n_chunks):
            swait(jj + lk - nbuf, (jj + lk) % nbuf)
            if jj + lk < n_chunks:
                gissue(jj + lk, (jj + lk) % nbuf)
            gwait(jj, jj % nbuf)
            sissue(jj, jj % nbuf)
        for cc in range(n_chunks - (nbuf - lk), n_chunks):
            swait(cc, cc % nbuf)

        plsc.subcore_barrier()
        pltpu.sync_copy(acc_sh.at[pl.ds(s * RPT, RPT)],
                        out_hbm.at[c].at[pl.ds(s * RPT, RPT)])

    return k(hw2, zeros, src_r, dst_r)


def _tc_scale(h, wcol):
    def body(h_ref, wc_ref, o_ref):
        hw = wc_ref[...] * h_ref[...]
        dh = hw.shape[1] // 2
        o_ref[0] = hw[:, :dh]
        o_ref[1] = hw[:, dh:]

    n, dim = h.shape
    return pl.pallas_call(
        body,
        out_shape=jax.ShapeDtypeStruct((2, n, dim // 2), jnp.float32),
    )(h, wcol)


def _tc_layer1(p, h, w_mat, b, wcol):
    def body(p_ref, h_ref, w_ref, b_ref, wc_ref, hn_ref, hwn_ref):
        x = jnp.concatenate([p_ref[0], p_ref[1]], axis=1) + h_ref[...]
        hn = jnp.maximum(
            jnp.dot(x, w_ref[...], preferred_element_type=jnp.float32)
            + b_ref[...], 0.0)
        hn_ref[...] = hn
        hwn_ref[...] = wc_ref[...] * hn

    d = w_mat.shape[1]
    return pl.pallas_call(
        body,
        out_shape=[
            jax.ShapeDtypeStruct((N_PAD, d), jnp.float32),
            jax.ShapeDtypeStruct((N_PAD, d), jnp.float32),
        ],
    )(p, h, w_mat, b, wcol)


def _tc_layer(p, h, w_mat, b, wcol):
    def body(p_ref, h_ref, w_ref, b_ref, wc_ref, hn_ref, hwn_ref):
        x = p_ref[0] + p_ref[1] + h_ref[...]
        hn = jnp.maximum(
            jnp.dot(x, w_ref[...], preferred_element_type=jnp.float32)
            + b_ref[...], 0.0)
        hn_ref[...] = hn
        hwn_ref[...] = wc_ref[...] * hn

    d = w_mat.shape[1]
    return pl.pallas_call(
        body,
        out_shape=[
            jax.ShapeDtypeStruct((N_PAD, d), jnp.float32),
            jax.ShapeDtypeStruct((N_PAD, d), jnp.float32),
        ],
    )(p, h, w_mat, b, wcol)


def _tc_last(p, h, w_mat, b):
    def body(p_ref, h_ref, w_ref, b_ref, o_ref):
        x = p_ref[0] + p_ref[1] + h_ref[...]
        o_ref[...] = jax.nn.sigmoid(
            jnp.dot(x, w_ref[...], preferred_element_type=jnp.float32)
            + b_ref[...])

    d = w_mat.shape[1]
    return pl.pallas_call(
        body,
        out_shape=jax.ShapeDtypeStruct((N_PAD, d), jnp.float32),
    )(p, h, w_mat, b)


def kernel(structure, H, input_weight, W1, b1, W2, b2, W3, b3):
    src = structure[0]
    dst = structure[1]
    pad = E_PAD - N_EDGES
    fill = jnp.full((pad,), N_NODES, jnp.int32)
    src_flat = jnp.concatenate([src, fill])
    dst_flat = jnp.concatenate([dst, fill])
    src_r = src_flat.reshape(NW, EPT_CHUNKS, CHUNK)
    dst_r = dst_flat.reshape(NW, EPT_CHUNKS, CHUNK)
    src_r16 = src_flat.reshape(NS, 2 * EPT_CHUNKS, CHUNK)
    dst_r16 = dst_flat.reshape(NS, 2 * EPT_CHUNKS, CHUNK)

    h_pad = jnp.zeros((N_PAD, D_IN), jnp.float32).at[:N_NODES].set(H)
    wcol = jnp.zeros((N_PAD, 1), jnp.float32).at[:N_NODES, 0].set(input_weight)
    z = jnp.zeros((N_PAD, D_IN), jnp.float32)

    hw1 = _tc_scale(h_pad, wcol)
    p1 = _sc_scatter_fsplit(hw1, z[:, :64], src_r16, dst_r16, 4, 2)
    h1, hw2 = _tc_layer1(p1, h_pad, W1, b1.reshape(1, -1), wcol)
    p2 = _sc_scatter(hw2, z[:, :32], src_r, dst_r, 32, 6, 3)
    h2, hw3 = _tc_layer(p2, h1, W2, b2.reshape(1, -1), wcol)
    p3 = _sc_scatter(hw3, z[:, :16], src_r, dst_r, 16, 6, 3)
    out = _tc_last(p3, h2, W3, b3.reshape(1, -1))
    return out[:N_NODES]
